# baseline (speedup 1.0000x reference)
import jax
import jax.numpy as jnp
from jax import lax
from jax.experimental import pallas as pl
from jax.experimental.pallas import tpu as pltpu, tpu_sc as plsc

E = 1_600_000
D = 16
OUT_D = 46
L = 16
B = 1000
NW = 32
CPW = E // (NW * B)
GRP = (B // L) * L
assert CPW * NW * B == E


def _sc_body(ea_hbm, wt_hbm, wr_hbm, out_hbm,
             wt_v, wr_v, ea_v, out_v, sin0, sin1, sout0, sout1):
    wid = lax.axis_index("s") * 2 + lax.axis_index("c")
    row0 = wid * (CPW * B)
    pltpu.sync_copy(wt_hbm, wt_v)
    pltpu.sync_copy(wr_hbm, wr_v)

    ins = (sin0, sin1)
    outs = (sout0, sout1)

    def in_copy(k, b):
        base = row0 + k * B
        return pltpu.make_async_copy(
            ea_hbm.at[pl.ds(base * D, B * D)], ea_v.at[b], ins[b])

    def out_copy(k, b):
        base = row0 + k * B
        return pltpu.make_async_copy(
            out_v.at[pl.ds(b * B * OUT_D, B * OUT_D)],
            out_hbm.at[pl.ds(base * OUT_D, B * OUT_D)], outs[b])

    def emit_edge(eb, ob, e):
        feat = eb[pl.ds(e * D, L)]
        fi = feat.astype(jnp.int32)
        t = fi[0]
        r = fi[1]
        o = (ob + e) * OUT_D
        out_v[pl.ds(o + 30, L)] = feat
        out_v[pl.ds(o + 16, L)] = wr_v[pl.ds(r * D, L)]
        out_v[pl.ds(o, L)] = wt_v[pl.ds(t * D, L)]

    def compute(b):
        eb = ea_v.at[b]
        ob = b * B

        @plsc.parallel_loop(0, GRP, step=L, unroll=2)
        def grp(i):
            for u in range(L):
                emit_edge(eb, ob, i + u)

        for e in range(GRP, B):
            emit_edge(eb, ob, e)

    in_copy(0, 0).start()
    in_copy(1, 1).start()

    def pair_body(i, carry):
        for b in (0, 1):
            k = 2 * i + b
            in_copy(k, b).wait()

            @pl.when(k >= 2)
            def _():
                out_copy(k, b).wait()

            compute(b)
            out_copy(k, b).start()

            @pl.when(k + 2 < CPW)
            def _():
                in_copy(k + 2, b).start()

        return carry

    lax.fori_loop(0, CPW // 2, pair_body, 0)

    out_copy(CPW - 2, 0).wait()
    out_copy(CPW - 1, 1).wait()


@jax.jit
def _run(ea_flat, wt_flat, wr_flat):
    mesh = plsc.VectorSubcoreMesh(core_axis_name="c", subcore_axis_name="s")
    f = pl.kernel(
        _sc_body,
        out_type=jax.ShapeDtypeStruct((E * OUT_D,), jnp.float32),
        mesh=mesh,
        scratch_types=[
            pltpu.VMEM((10 * D,), jnp.float32),
            pltpu.VMEM((10 * D,), jnp.float32),
            pltpu.VMEM((2, B * D), jnp.float32),
            pltpu.VMEM((2 * B * OUT_D,), jnp.float32),
            pltpu.SemaphoreType.DMA,
            pltpu.SemaphoreType.DMA,
            pltpu.SemaphoreType.DMA,
            pltpu.SemaphoreType.DMA,
        ],
    )
    return f(ea_flat, wt_flat, wr_flat).reshape(E, OUT_D)


def kernel(edge_attr, W_type, W_ring):
    return _run(edge_attr.reshape(E * D),
                W_type.reshape(10 * D),
                W_ring.reshape(10 * D))


# speedup vs baseline: 1.5159x; 1.5159x over previous
import jax
import jax.numpy as jnp
from jax import lax
from jax.experimental import pallas as pl
from jax.experimental.pallas import tpu as pltpu, tpu_sc as plsc

E = 1_600_000
D = 16
OUT_D = 46
L = 16
B = 400
NW = 32
CPW = E // (NW * B)
assert CPW * NW * B == E and B % L == 0


def _sc_body(ea_hbm, wt_hbm, wr_hbm, out_hbm,
             wt_v, wr_v, ea_v, out_v, sin0, sin1, sout0, sout1):
    wid = lax.axis_index("s") * 2 + lax.axis_index("c")
    row0 = wid * (CPW * B)
    pltpu.sync_copy(wt_hbm, wt_v)
    pltpu.sync_copy(wr_hbm, wr_v)

    ins = (sin0, sin1)
    outs = (sout0, sout1)

    def in_copy(k, b):
        base = row0 + k * B
        return pltpu.make_async_copy(
            ea_hbm.at[pl.ds(base * D, B * D)], ea_v.at[b], ins[b])

    def out_copy(k, b):
        base = row0 + k * B
        return pltpu.make_async_copy(
            out_v.at[pl.ds(b * B, B)], out_hbm.at[pl.ds(base, B)], outs[b])

    def emit_edge(eb, ob, e):
        feat = eb[pl.ds(e * D, L)]
        fi = feat.astype(jnp.int32)
        t = fi[0]
        r = fi[1]
        o = ob + e
        out_v[o, pl.ds(30, L)] = feat
        out_v[o, pl.ds(16, L)] = wr_v[pl.ds(r * D, L)]
        out_v[o, pl.ds(0, L)] = wt_v[pl.ds(t * D, L)]

    def compute(b):
        eb = ea_v.at[b]
        ob = b * B

        @plsc.parallel_loop(0, B, step=L, unroll=2)
        def grp(i):
            for u in range(L):
                emit_edge(eb, ob, i + u)

    def do_chunk(k, b):
        in_copy(k, b).wait()

        @pl.when(k >= 2)
        def _():
            out_copy(k, b).wait()

        compute(b)
        out_copy(k, b).start()

        @pl.when(k + 2 < CPW)
        def _():
            in_copy(k + 2, b).start()

    in_copy(0, 0).start()
    in_copy(1, 1).start()

    def pair_body(i, carry):
        do_chunk(2 * i, 0)
        do_chunk(2 * i + 1, 1)
        return carry

    lax.fori_loop(0, CPW // 2, pair_body, 0)

    in_copy(CPW - 1, 0).wait()
    out_copy(CPW - 1, 0).wait()
    compute(0)
    out_copy(CPW - 1, 0).start()

    out_copy(CPW - 2, 1).wait()
    out_copy(CPW - 1, 0).wait()


@jax.jit
def _run(ea_flat, wt_flat, wr_flat):
    mesh = plsc.VectorSubcoreMesh(core_axis_name="c", subcore_axis_name="s")
    f = pl.kernel(
        _sc_body,
        out_type=jax.ShapeDtypeStruct((E, OUT_D), jnp.float32),
        mesh=mesh,
        scratch_types=[
            pltpu.VMEM((10 * D,), jnp.float32),
            pltpu.VMEM((10 * D,), jnp.float32),
            pltpu.VMEM((2, B * D), jnp.float32),
            pltpu.VMEM((2 * B, OUT_D), jnp.float32),
            pltpu.SemaphoreType.DMA,
            pltpu.SemaphoreType.DMA,
            pltpu.SemaphoreType.DMA,
            pltpu.SemaphoreType.DMA,
        ],
    )
    return f(ea_flat, wt_flat, wr_flat)


def kernel(edge_attr, W_type, W_ring):
    return _run(edge_attr.reshape(E * D),
                W_type.reshape(10 * D),
                W_ring.reshape(10 * D))


# speedup vs baseline: 9.4542x; 6.2369x over previous
import jax
import jax.numpy as jnp
from jax import lax
from jax.experimental import pallas as pl
from jax.experimental.pallas import tpu as pltpu, tpu_sc as plsc

E = 1_600_000
D = 16
OUT_D = 46
L = 16
C = 512
NW = 32
N_CHUNKS = E // C
CPW = -(-N_CHUNKS // NW)
GRPS = C // L

_DNUMS = lax.GatherDimensionNumbers(
    offset_dims=(), collapsed_slice_dims=(0,), start_index_map=(0,))


def _lut16(table_row, idx):
    return lax.gather(table_row, idx.reshape(L, 1), _DNUMS, (1,),
                      mode=lax.GatherScatterMode.PROMISE_IN_BOUNDS)


def _sc_body(ea_hbm, tc_hbm, out_hbm,
             tc_v, in0, in1, st0, st1, sin0, sin1, sout0, sout1):
    wid = lax.axis_index("s") * 2 + lax.axis_index("c")
    pltpu.sync_copy(tc_hbm, tc_v)

    ins = (sin0, sin1)
    outs = (sout0, sout1)
    inbufs = (in0, in1)
    stages = (st0, st1)

    def in_copy(g, b):
        return pltpu.make_async_copy(
            ea_hbm.at[:, pl.ds(g * C, C)], inbufs[b], ins[b])

    def out_copy(g, b):
        return pltpu.make_async_copy(
            stages[b], out_hbm.at[:, pl.ds(g * C, C)], outs[b])

    def compute(b):
        ib = inbufs[b]
        sb = stages[b]

        @plsc.parallel_loop(0, GRPS, step=1, unroll=2)
        def grp(i):
            goff = i * L
            ti = ib[0, pl.ds(goff, L)].astype(jnp.int32)
            ri = ib[1, pl.ds(goff, L)].astype(jnp.int32)
            for j in range(D):
                sb[j, pl.ds(goff, L)] = _lut16(tc_v[j, :], ti)
            for j in range(D):
                sb[D + j, pl.ds(goff, L)] = _lut16(tc_v[D + j, :], ri)
            for c in range(2, D):
                sb[30 + c, pl.ds(goff, L)] = ib[c, pl.ds(goff, L)]

    def do_chunk(k, b):
        g = k * NW + wid

        @pl.when(g < N_CHUNKS)
        def _():
            in_copy(g, b).wait()

            @pl.when(k >= 2)
            def _():
                out_copy(g, b).wait()

            compute(b)
            out_copy(g, b).start()

            @pl.when(g + 2 * NW < N_CHUNKS)
            def _():
                in_copy(g + 2 * NW, b).start()

    @pl.when(wid < N_CHUNKS)
    def _():
        in_copy(wid, 0).start()

    @pl.when(NW + wid < N_CHUNKS)
    def _():
        in_copy(NW + wid, 1).start()

    def pair_body(i, carry):
        do_chunk(2 * i, 0)
        do_chunk(2 * i + 1, 1)
        return carry

    lax.fori_loop(0, CPW // 2, pair_body, 0)

    out_copy(wid, 0).wait()
    out_copy(wid, 1).wait()


@jax.jit
def _run(ea_t, tcols):
    mesh = plsc.VectorSubcoreMesh(core_axis_name="c", subcore_axis_name="s")
    f = pl.kernel(
        _sc_body,
        out_type=jax.ShapeDtypeStruct((OUT_D, E), jnp.float32),
        mesh=mesh,
        scratch_types=[
            pltpu.VMEM((2 * D, L), jnp.float32),
            pltpu.VMEM((D, C), jnp.float32),
            pltpu.VMEM((D, C), jnp.float32),
            pltpu.VMEM((OUT_D, C), jnp.float32),
            pltpu.VMEM((OUT_D, C), jnp.float32),
            pltpu.SemaphoreType.DMA,
            pltpu.SemaphoreType.DMA,
            pltpu.SemaphoreType.DMA,
            pltpu.SemaphoreType.DMA,
        ],
    )
    return f(ea_t, tcols)


def kernel(edge_attr, W_type, W_ring):
    tcols = jnp.concatenate([
        jnp.pad(W_type.T, ((0, 0), (0, L - 10))),
        jnp.pad(W_ring.T, ((0, 0), (0, L - 10))),
    ], axis=0)
    return _run(edge_attr.T, tcols).T
